# baseline (speedup 1.0000x reference)
import functools

import jax
import jax.numpy as jnp
from jax import lax
from jax.experimental import pallas as pl
from jax.experimental.pallas import tpu as pltpu
from jax.experimental.pallas import tpu_sc as plsc

NC, NS = 2, 16
NW = NC * NS
CHUNK = 1024
IDX_PER_STREAM = 128


def kernel(x, W):
    B = x.size
    D = W.shape[1]
    xf = x.reshape(B)

    b_per_w = B // NW
    n_iter = b_per_w // CHUNK
    n_streams = CHUNK // IDX_PER_STREAM

    mesh = plsc.VectorSubcoreMesh(core_axis_name="c", subcore_axis_name="s")

    @functools.partial(
        pl.kernel,
        out_type=jax.ShapeDtypeStruct((B, D), jnp.float32),
        mesh=mesh,
        scratch_types=[
            pltpu.VMEM((CHUNK,), jnp.int32),
            pltpu.VMEM((CHUNK, D), jnp.float32),
            pltpu.SemaphoreType.DMA,
        ],
        compiler_params=pltpu.CompilerParams(use_tc_tiling_on_sc=False),
    )
    def emb(x_hbm, w_hbm, out_hbm, idx_v, rows_v, sem):
        wid = lax.axis_index("s") * NC + lax.axis_index("c")
        base = wid * b_per_w

        def body(it, _):
            off = pl.multiple_of(base + it * CHUNK, CHUNK)
            pltpu.sync_copy(x_hbm.at[pl.ds(off, CHUNK)], idx_v)
            copies = [
                pltpu.make_async_copy(
                    w_hbm.at[idx_v.at[pl.ds(j * IDX_PER_STREAM, IDX_PER_STREAM)]],
                    rows_v.at[pl.ds(j * IDX_PER_STREAM, IDX_PER_STREAM)],
                    sem,
                )
                for j in range(n_streams)
            ]
            for c in copies:
                c.start()
            for c in copies:
                c.wait()
            pltpu.sync_copy(rows_v, out_hbm.at[pl.ds(off, CHUNK)])
            return ()

        lax.fori_loop(0, n_iter, body, ())

    out = emb(xf, W)
    return out.reshape(*x.shape, D)


# speedup vs baseline: 6.6997x; 6.6997x over previous
import functools

import jax
import jax.numpy as jnp
from jax import lax
from jax.experimental import pallas as pl
from jax.experimental.pallas import tpu as pltpu
from jax.experimental.pallas import tpu_sc as plsc

NC, NS = 2, 16
NW = NC * NS
CHUNK = 512
IDX_PER_STREAM = 128
NBUF = 2


def kernel(x, W):
    B = x.size
    D = W.shape[1]
    xf = x.reshape(B)

    b_per_w = B // NW
    n_iter = b_per_w // CHUNK
    n_streams = CHUNK // IDX_PER_STREAM

    mesh = plsc.VectorSubcoreMesh(core_axis_name="c", subcore_axis_name="s")

    @functools.partial(
        pl.kernel,
        out_type=jax.ShapeDtypeStruct((B, D), jnp.float32),
        mesh=mesh,
        scratch_types=[
            pltpu.VMEM_SHARED((10, D), jnp.float32),
            pltpu.VMEM((NBUF, CHUNK), jnp.int32),
            pltpu.VMEM((NBUF, CHUNK, D), jnp.float32),
            pltpu.SemaphoreType.DMA,
            pltpu.SemaphoreType.DMA,
            pltpu.SemaphoreType.DMA,
            pltpu.SemaphoreType.DMA,
            pltpu.SemaphoreType.DMA,
        ],
        compiler_params=pltpu.CompilerParams(use_tc_tiling_on_sc=False),
    )
    def emb(x_hbm, w_hbm, out_hbm, w_sh, idx_v, rows_v, gsem, isem0, isem1,
            osem0, osem1):
        sid = lax.axis_index("s")
        wid = sid * NC + lax.axis_index("c")
        base = wid * b_per_w
        isems = (isem0, isem1)
        osems = (osem0, osem1)

        @pl.when(sid == 0)
        def _():
            pltpu.sync_copy(w_hbm, w_sh)
        plsc.subcore_barrier()

        def idx_in(it, b):
            off = pl.multiple_of(base + it * CHUNK, CHUNK)
            return pltpu.make_async_copy(
                x_hbm.at[pl.ds(off, CHUNK)], idx_v.at[b], isems[b])

        def rows_out(it, b):
            off = pl.multiple_of(base + it * CHUNK, CHUNK)
            return pltpu.make_async_copy(
                rows_v.at[b], out_hbm.at[pl.ds(off, CHUNK)], osems[b])

        for b in range(NBUF):
            idx_in(b, b).start()

        def half(it, b):
            @pl.when(it >= NBUF)
            def _():
                rows_out(it - NBUF, b).wait()
            idx_in(it, b).wait()
            copies = [
                pltpu.make_async_copy(
                    w_sh.at[idx_v.at[b].at[pl.ds(j * IDX_PER_STREAM,
                                                 IDX_PER_STREAM)]],
                    rows_v.at[b].at[pl.ds(j * IDX_PER_STREAM, IDX_PER_STREAM)],
                    gsem,
                )
                for j in range(n_streams)
            ]
            for c in copies:
                c.start()
            for c in copies:
                c.wait()
            rows_out(it, b).start()
            @pl.when(it + NBUF < n_iter)
            def _():
                idx_in(it + NBUF, b).start()

        def body(i2, _):
            it = i2 * NBUF
            for b in range(NBUF):
                half(it + b, b)
            return ()

        lax.fori_loop(0, n_iter // NBUF, body, ())
        for b in range(NBUF):
            rows_out(n_iter - NBUF + b, b).wait()

    out = emb(xf, W)
    return out.reshape(*x.shape, D)


# speedup vs baseline: 6.7218x; 1.0033x over previous
import functools

import jax
import jax.numpy as jnp
from jax import lax
from jax.experimental import pallas as pl
from jax.experimental.pallas import tpu as pltpu
from jax.experimental.pallas import tpu_sc as plsc

NC, NS = 2, 16
NW = NC * NS
CHUNK = 512
IDX_PER_STREAM = 128
NBUF = 2


def kernel(x, W):
    B = x.size
    D = W.shape[1]
    xf = x.reshape(B)

    b_per_w = B // NW
    n_iter = b_per_w // CHUNK
    n_streams = CHUNK // IDX_PER_STREAM

    mesh = plsc.VectorSubcoreMesh(core_axis_name="c", subcore_axis_name="s")

    @functools.partial(
        pl.kernel,
        out_type=jax.ShapeDtypeStruct((B, D), jnp.float32),
        mesh=mesh,
        scratch_types=[
            pltpu.VMEM_SHARED((10, D), jnp.float32),
            pltpu.VMEM((NBUF, CHUNK), jnp.int32),
            pltpu.VMEM((NBUF, CHUNK, D), jnp.float32),
            pltpu.SemaphoreType.DMA,
            pltpu.SemaphoreType.DMA,
            pltpu.SemaphoreType.DMA,
            pltpu.SemaphoreType.DMA,
            pltpu.SemaphoreType.DMA,
            pltpu.SemaphoreType.DMA,
        ],
        compiler_params=pltpu.CompilerParams(use_tc_tiling_on_sc=False),
    )
    def emb(x_hbm, w_hbm, out_hbm, w_sh, idx_v, rows_v, gsem0, gsem1,
            isem0, isem1, osem0, osem1):
        sid = lax.axis_index("s")
        wid = sid * NC + lax.axis_index("c")
        base = wid * b_per_w
        gsems = (gsem0, gsem1)
        isems = (isem0, isem1)
        osems = (osem0, osem1)

        @pl.when(sid == 0)
        def _():
            pltpu.sync_copy(w_hbm, w_sh)
        plsc.subcore_barrier()

        def idx_in(it, b):
            off = pl.multiple_of(base + it * CHUNK, CHUNK)
            return pltpu.make_async_copy(
                x_hbm.at[pl.ds(off, CHUNK)], idx_v.at[b], isems[b])

        def rows_out(it, b):
            off = pl.multiple_of(base + it * CHUNK, CHUNK)
            return pltpu.make_async_copy(
                rows_v.at[b], out_hbm.at[pl.ds(off, CHUNK)], osems[b])

        def gathers(b):
            return [
                pltpu.make_async_copy(
                    w_sh.at[idx_v.at[b].at[pl.ds(j * IDX_PER_STREAM,
                                                 IDX_PER_STREAM)]],
                    rows_v.at[b].at[pl.ds(j * IDX_PER_STREAM, IDX_PER_STREAM)],
                    gsems[b],
                )
                for j in range(n_streams)
            ]

        for b in range(NBUF):
            idx_in(b, b).start()

        def half(it, b):
            @pl.when(it >= NBUF)
            def _():
                rows_out(it - NBUF, b).wait()
            idx_in(it, b).wait()
            for c in gathers(b):
                c.start()
            @pl.when(it >= 1)
            def _():
                for c in gathers(1 - b):
                    c.wait()
                rows_out(it - 1, 1 - b).start()
                @pl.when(it + 1 < n_iter)
                def _():
                    idx_in(it + 1, 1 - b).start()

        def body(i2, _):
            it = i2 * NBUF
            for b in range(NBUF):
                half(it + b, b)
            return ()

        lax.fori_loop(0, n_iter // NBUF, body, ())
        last_b = (n_iter - 1) % NBUF
        for c in gathers(last_b):
            c.wait()
        rows_out(n_iter - 1, last_b).start()
        rows_out(n_iter - 2, 1 - last_b).wait()
        rows_out(n_iter - 1, last_b).wait()

    out = emb(xf, W)
    return out.reshape(*x.shape, D)
